# baseline (speedup 1.0000x reference)
import functools

import jax
import jax.numpy as jnp
from jax import lax
from jax.experimental import pallas as pl
from jax.experimental.pallas import tpu as pltpu
from jax.experimental.pallas import tpu_sc as plsc

NUM_FIELDS = 26
VOCAB = 100000
DIM = 16
BATCH = 16384

_INFO = plsc.get_sparse_core_info()
_NC = _INFO.num_cores
_NS = _INFO.num_subcores
_NW = _NC * _NS
_CHUNK = BATCH // _NW
_GSUB = 128
_NGATHER = _CHUNK // _GSUB


def _body(*refs):
    idx_refs = refs[:NUM_FIELDS]
    tab_refs = refs[NUM_FIELDS:2 * NUM_FIELDS]
    out_ref = refs[2 * NUM_FIELDS]
    idx_v = refs[2 * NUM_FIELDS + 1]
    rows_v = refs[2 * NUM_FIELDS + 2]
    idx_sem = refs[2 * NUM_FIELDS + 3]
    gat_sem = refs[2 * NUM_FIELDS + 4]
    out_sem = refs[2 * NUM_FIELDS + 5]

    wid = lax.axis_index("s") * _NC + lax.axis_index("c")
    base = wid * _CHUNK

    pltpu.async_copy(idx_refs[0].at[pl.ds(base, _CHUNK)], idx_v.at[0],
                     idx_sem)

    for f in range(NUM_FIELDS):
        cur = f % 2
        nxt = (f + 1) % 2
        pltpu.make_async_copy(idx_refs[f].at[pl.ds(base, _CHUNK)],
                              idx_v.at[cur], idx_sem).wait()
        if f + 1 < NUM_FIELDS:
            pltpu.async_copy(idx_refs[f + 1].at[pl.ds(base, _CHUNK)],
                             idx_v.at[nxt], idx_sem)

        if f >= 2:
            pltpu.make_async_copy(
                rows_v.at[cur],
                out_ref.at[pl.ds(base, _CHUNK), f - 2],
                out_sem).wait()
        for j in range(_NGATHER):
            pltpu.async_copy(
                tab_refs[f].at[idx_v.at[cur, pl.ds(j * _GSUB, _GSUB)]],
                rows_v.at[cur, pl.ds(j * _GSUB, _GSUB)],
                gat_sem)
        for j in range(_NGATHER):
            pltpu.make_async_copy(
                tab_refs[f].at[idx_v.at[cur, pl.ds(j * _GSUB, _GSUB)]],
                rows_v.at[cur, pl.ds(j * _GSUB, _GSUB)],
                gat_sem).wait()

        pltpu.async_copy(rows_v.at[cur],
                         out_ref.at[pl.ds(base, _CHUNK), f],
                         out_sem)

    for f in (NUM_FIELDS - 2, NUM_FIELDS - 1):
        pltpu.make_async_copy(rows_v.at[f % 2],
                              out_ref.at[pl.ds(base, _CHUNK), f],
                              out_sem).wait()


@functools.partial(jax.jit)
def _cat_embed(idx_list, tab_list):
    run = pl.kernel(
        _body,
        mesh=plsc.VectorSubcoreMesh(core_axis_name="c", subcore_axis_name="s"),
        compiler_params=pltpu.CompilerParams(use_tc_tiling_on_sc=False),
        out_type=jax.ShapeDtypeStruct((BATCH, NUM_FIELDS, DIM), jnp.float32),
        scratch_types=[
            pltpu.VMEM((2, _CHUNK), jnp.int32),
            pltpu.VMEM((2, _CHUNK, DIM), jnp.float32),
            pltpu.SemaphoreType.DMA,
            pltpu.SemaphoreType.DMA,
            pltpu.SemaphoreType.DMA,
        ],
    )
    out = run(*idx_list, *tab_list)
    return out.reshape(BATCH, NUM_FIELDS * DIM)


def kernel(f00, f01, f02, f03, f04, f05, f06, f07, f08, f09,
           f10, f11, f12, f13, f14, f15, f16, f17, f18, f19,
           f20, f21, f22, f23, f24, f25,
           W_f00, W_f01, W_f02, W_f03, W_f04, W_f05, W_f06, W_f07,
           W_f08, W_f09, W_f10, W_f11, W_f12, W_f13, W_f14, W_f15,
           W_f16, W_f17, W_f18, W_f19, W_f20, W_f21, W_f22, W_f23,
           W_f24, W_f25):
    idx = [f00, f01, f02, f03, f04, f05, f06, f07, f08, f09,
           f10, f11, f12, f13, f14, f15, f16, f17, f18, f19,
           f20, f21, f22, f23, f24, f25]
    tabs = [W_f00, W_f01, W_f02, W_f03, W_f04, W_f05, W_f06, W_f07,
            W_f08, W_f09, W_f10, W_f11, W_f12, W_f13, W_f14, W_f15,
            W_f16, W_f17, W_f18, W_f19, W_f20, W_f21, W_f22, W_f23,
            W_f24, W_f25]
    idx = [i.astype(jnp.int32) for i in idx]
    return _cat_embed(idx, tabs)


# speedup vs baseline: 1.0062x; 1.0062x over previous
import functools

import jax
import jax.numpy as jnp
from jax import lax
from jax.experimental import pallas as pl
from jax.experimental.pallas import tpu as pltpu
from jax.experimental.pallas import tpu_sc as plsc

NF = 26
VOCAB = 100000
DIM = 16
BATCH = 16384
PACK = 128 // DIM
NROWS = VOCAB // PACK
LANES = 16

_INFO = plsc.get_sparse_core_info()
_NC = _INFO.num_cores
_NS = _INFO.num_subcores
_NW = _NC * _NS
_BPW = BATCH // _NW
_SUB = 128
_NSUB = _BPW // _SUB


def _body(rowidx_hbm, off_hbm, wstack_hbm, out_hbm,
          ridx_all, off_all, rows_v, stag,
          idx_sem, gat_sem0, gat_sem1, out_sem0, out_sem1):
    wid = lax.axis_index("s") * _NC + lax.axis_index("c")
    iota = lax.iota(jnp.int32, LANES)

    def do_subchunk(sb, _):
        base = wid * _BPW + sb * _SUB

        pltpu.async_copy(rowidx_hbm.at[:, pl.ds(base, _SUB)], ridx_all,
                         idx_sem)
        pltpu.make_async_copy(rowidx_hbm.at[:, pl.ds(base, _SUB)], ridx_all,
                              idx_sem).wait()
        pltpu.async_copy(off_hbm.at[:, pl.ds(base, _SUB)], off_all, idx_sem)
        pltpu.make_async_copy(off_hbm.at[:, pl.ds(base, _SUB)], off_all,
                              idx_sem).wait()

        def gather_of(f, par):
            return pltpu.make_async_copy(
                wstack_hbm.at[f].at[ridx_all.at[f]],
                rows_v.at[par], gat_sem0)

        def gather_of1(f, par):
            return pltpu.make_async_copy(
                wstack_hbm.at[f].at[ridx_all.at[f]],
                rows_v.at[par], gat_sem1)

        def out_of(f, par, sem):
            return pltpu.make_async_copy(
                stag.at[par],
                out_hbm.at[pl.ds(f * DIM, DIM), pl.ds(base, _SUB)],
                sem)

        gather_of(0, 0).start()

        def do_field(f, _):
            par = lax.rem(f, 2)

            @pl.when(jnp.logical_and(f + 1 < NF, par == 0))
            def _():
                gather_of1(f + 1, 1).start()

            @pl.when(jnp.logical_and(f + 1 < NF, par == 1))
            def _():
                gather_of(f + 1, 0).start()

            @pl.when(par == 0)
            def _():
                gather_of(f, 0).wait()

            @pl.when(par == 1)
            def _():
                gather_of1(f, 1).wait()

            @pl.when(jnp.logical_and(f >= 2, par == 0))
            def _():
                out_of(f - 2, 0, out_sem0).wait()

            @pl.when(jnp.logical_and(f >= 2, par == 1))
            def _():
                out_of(f - 2, 1, out_sem1).wait()

            for g in range(_SUB // LANES):
                i_vec = iota + (g * LANES)
                o_vec = off_all[f, pl.ds(g * LANES, LANES)]
                for d in range(DIM):
                    vals = plsc.load_gather(rows_v.at[par],
                                            [i_vec, o_vec + d])
                    stag[par, d, pl.ds(g * LANES, LANES)] = vals

            @pl.when(par == 0)
            def _():
                out_of(f, 0, out_sem0).start()

            @pl.when(par == 1)
            def _():
                out_of(f, 1, out_sem1).start()

            return ()

        lax.fori_loop(0, NF, do_field, (), unroll=False)

        out_of(NF - 2, 0, out_sem0).wait()
        out_of(NF - 1, 1, out_sem1).wait()
        return ()

    lax.fori_loop(0, _NSUB, do_subchunk, (), unroll=False)


@jax.jit
def _cat_embed(idx_list, tab_list):
    idxstack = jnp.stack(idx_list)
    rowidx = idxstack >> 3
    off = (idxstack & 7) << 4
    wstack = jnp.stack([w.reshape(NROWS, PACK * DIM) for w in tab_list])

    run = pl.kernel(
        _body,
        mesh=plsc.VectorSubcoreMesh(core_axis_name="c", subcore_axis_name="s"),
        out_type=jax.ShapeDtypeStruct((NF * DIM, BATCH), jnp.float32),
        compiler_params=pltpu.CompilerParams(needs_layout_passes=False),
        scratch_types=[
            pltpu.VMEM((NF, _SUB), jnp.int32),
            pltpu.VMEM((NF, _SUB), jnp.int32),
            pltpu.VMEM((2, _SUB, PACK * DIM), jnp.float32),
            pltpu.VMEM((2, DIM, _SUB), jnp.float32),
            pltpu.SemaphoreType.DMA,
            pltpu.SemaphoreType.DMA,
            pltpu.SemaphoreType.DMA,
            pltpu.SemaphoreType.DMA,
            pltpu.SemaphoreType.DMA,
        ],
    )
    out_t = run(rowidx, off, wstack)
    return out_t.T


def kernel(f00, f01, f02, f03, f04, f05, f06, f07, f08, f09,
           f10, f11, f12, f13, f14, f15, f16, f17, f18, f19,
           f20, f21, f22, f23, f24, f25,
           W_f00, W_f01, W_f02, W_f03, W_f04, W_f05, W_f06, W_f07,
           W_f08, W_f09, W_f10, W_f11, W_f12, W_f13, W_f14, W_f15,
           W_f16, W_f17, W_f18, W_f19, W_f20, W_f21, W_f22, W_f23,
           W_f24, W_f25):
    idx = [f00, f01, f02, f03, f04, f05, f06, f07, f08, f09,
           f10, f11, f12, f13, f14, f15, f16, f17, f18, f19,
           f20, f21, f22, f23, f24, f25]
    tabs = [W_f00, W_f01, W_f02, W_f03, W_f04, W_f05, W_f06, W_f07,
            W_f08, W_f09, W_f10, W_f11, W_f12, W_f13, W_f14, W_f15,
            W_f16, W_f17, W_f18, W_f19, W_f20, W_f21, W_f22, W_f23,
            W_f24, W_f25]
    idx = [i.astype(jnp.int32) for i in idx]
    return _cat_embed(idx, tabs)


# speedup vs baseline: 2.0863x; 2.0734x over previous
import functools

import jax
import jax.numpy as jnp
from jax import lax
from jax.experimental import pallas as pl
from jax.experimental.pallas import tpu as pltpu
from jax.experimental.pallas import tpu_sc as plsc

NF = 26
VOCAB = 100000
DIM = 16
BATCH = 16384
PACK = 128 // DIM
LANES = 16

CH_COLS = 1408
NCH = 71
CH_ROWS = CH_COLS // PACK
NP = 12504

_INFO = plsc.get_sparse_core_info()
_NC = _INFO.num_cores
_NS = _INFO.num_subcores
_NW = _NC * _NS
_BPW = BATCH // _NW
_SUB = 128
_NSUB = _BPW // _SUB


def _pack_body(*refs):
    wt_refs = refs[:NF]
    tail_ref = refs[NF]
    out_ref = refs[NF + 1]
    lbuf = refs[NF + 2]
    stag = refs[NF + 3]
    lsem0, lsem1, wsem0, wsem1 = refs[NF + 4:NF + 8]

    wid = lax.axis_index("s") * _NC + lax.axis_index("c")
    iota = lax.iota(jnp.int32, LANES)
    half = iota >> 3
    ccol = [(iota & 7) * DIM + d for d in range(DIM)]

    def load_of(f, c, par, sem):
        col0 = c * CH_COLS
        return (
            pltpu.make_async_copy(
                wt_refs[f].at[pl.ds(0, 8), pl.ds(col0, CH_COLS)],
                lbuf.at[par, pl.ds(0, 8)], sem),
            pltpu.make_async_copy(
                wt_refs[f].at[pl.ds(8, 8), pl.ds(col0, CH_COLS)],
                lbuf.at[par, pl.ds(8, 8)], sem),
        )

    def write_of(f, c, par, sem):
        return pltpu.make_async_copy(
            stag.at[par], out_ref.at[f, pl.ds(c * CH_ROWS, CH_ROWS)], sem)

    for f in range(NF):
        base_c = lax.rem(wid + (7 * f) % 32, _NW)
        has3 = base_c < NCH - 2 * _NW

        for cp in load_of(f, base_c, 0, lsem0):
            cp.start()

        def do_rep(rep, _, f=f, base_c=base_c, has3=has3):
            c = base_c + rep * _NW
            active = jnp.logical_or(rep < 2, has3)

            @pl.when(active)
            def _():
                par = lax.rem(rep, 2)
                nxt_active = jnp.logical_or(rep == 0,
                                            jnp.logical_and(rep == 1, has3))

                @pl.when(jnp.logical_and(nxt_active, par == 0))
                def _():
                    for cp in load_of(f, c + _NW, 1, lsem1):
                        cp.start()

                @pl.when(jnp.logical_and(nxt_active, par == 1))
                def _():
                    for cp in load_of(f, c + _NW, 0, lsem0):
                        cp.start()

                @pl.when(par == 0)
                def _():
                    for cp in load_of(f, c, 0, lsem0):
                        cp.wait()

                @pl.when(par == 1)
                def _():
                    for cp in load_of(f, c, 1, lsem1):
                        cp.wait()

                @pl.when(jnp.logical_and(rep >= 2, par == 0))
                def _():
                    write_of(f, c - 2 * _NW, 0, wsem0).wait()

                def do_group(g, _, par=par):
                    r_vec = half + 2 * g
                    for d in range(DIM):
                        row = lbuf[par, d, pl.ds(g * LANES, LANES)]
                        plsc.store_scatter(stag.at[par], [r_vec, ccol[d]],
                                           row)
                    return ()

                lax.fori_loop(0, CH_COLS // LANES, do_group, (),
                              unroll=False)

                @pl.when(par == 0)
                def _():
                    write_of(f, c, 0, wsem0).start()

                @pl.when(par == 1)
                def _():
                    write_of(f, c, 1, wsem1).start()

            return ()

        lax.fori_loop(0, 3, do_rep, (), unroll=False)

        @pl.when(jnp.logical_not(has3))
        def _():
            write_of(f, base_c, 0, wsem0).wait()

        write_of(f, base_c + _NW, 1, wsem1).wait()

        @pl.when(has3)
        def _():
            write_of(f, base_c + 2 * _NW, 0, wsem0).wait()

    @pl.when(wid == 0)
    def _():
        for f in range(NF):
            pltpu.sync_copy(tail_ref.at[f], out_ref.at[f, pl.ds(NP - 8, 8)])


def _gather_body(rowidx_hbm, off_hbm, wstack_hbm, out_hbm,
                 ridx_all, off_all, rows_v, stag,
                 idx_sem, gat_sem0, gat_sem1, out_sem0, out_sem1):
    wid = lax.axis_index("s") * _NC + lax.axis_index("c")
    iota = lax.iota(jnp.int32, LANES)

    def do_subchunk(sb, _):
        base = wid * _BPW + sb * _SUB

        pltpu.async_copy(rowidx_hbm.at[:, pl.ds(base, _SUB)], ridx_all,
                         idx_sem)
        pltpu.make_async_copy(rowidx_hbm.at[:, pl.ds(base, _SUB)], ridx_all,
                              idx_sem).wait()
        pltpu.async_copy(off_hbm.at[:, pl.ds(base, _SUB)], off_all, idx_sem)
        pltpu.make_async_copy(off_hbm.at[:, pl.ds(base, _SUB)], off_all,
                              idx_sem).wait()

        def gather_of(f, par, sem):
            return pltpu.make_async_copy(
                wstack_hbm.at[f].at[ridx_all.at[f]],
                rows_v.at[par], sem)

        def out_of(f, par, sem):
            return pltpu.make_async_copy(
                stag.at[par],
                out_hbm.at[pl.ds(f * DIM, DIM), pl.ds(base, _SUB)],
                sem)

        gather_of(0, 0, gat_sem0).start()

        def do_field(f, _):
            par = lax.rem(f, 2)

            @pl.when(jnp.logical_and(f + 1 < NF, par == 0))
            def _():
                gather_of(f + 1, 1, gat_sem1).start()

            @pl.when(jnp.logical_and(f + 1 < NF, par == 1))
            def _():
                gather_of(f + 1, 0, gat_sem0).start()

            @pl.when(par == 0)
            def _():
                gather_of(f, 0, gat_sem0).wait()

            @pl.when(par == 1)
            def _():
                gather_of(f, 1, gat_sem1).wait()

            @pl.when(jnp.logical_and(f >= 2, par == 0))
            def _():
                out_of(f - 2, 0, out_sem0).wait()

            @pl.when(jnp.logical_and(f >= 2, par == 1))
            def _():
                out_of(f - 2, 1, out_sem1).wait()

            for g in range(_SUB // LANES):
                i_vec = iota + (g * LANES)
                o_vec = off_all[f, pl.ds(g * LANES, LANES)]
                for d in range(DIM):
                    vals = plsc.load_gather(rows_v.at[par],
                                            [i_vec, o_vec + d])
                    stag[par, d, pl.ds(g * LANES, LANES)] = vals

            @pl.when(par == 0)
            def _():
                out_of(f, 0, out_sem0).start()

            @pl.when(par == 1)
            def _():
                out_of(f, 1, out_sem1).start()

            return ()

        lax.fori_loop(0, NF, do_field, (), unroll=False)

        out_of(NF - 2, 0, out_sem0).wait()
        out_of(NF - 1, 1, out_sem1).wait()
        return ()

    lax.fori_loop(0, _NSUB, do_subchunk, (), unroll=False)


@jax.jit
def _cat_embed(idx_list, tab_list):
    idxstack = jnp.stack(idx_list)
    rowidx = idxstack >> 3
    off = (idxstack & 7) << 4

    wt_list = [w.T for w in tab_list]
    tail = jnp.stack([
        jnp.pad(w[VOCAB - 32:], ((0, 32), (0, 0))).reshape(8, 128)
        for w in tab_list])

    mesh = plsc.VectorSubcoreMesh(core_axis_name="c", subcore_axis_name="s")
    params = pltpu.CompilerParams(needs_layout_passes=False)

    pack = pl.kernel(
        _pack_body,
        mesh=mesh,
        out_type=jax.ShapeDtypeStruct((NF, NP, PACK * DIM), jnp.float32),
        compiler_params=params,
        scratch_types=[
            pltpu.VMEM((2, DIM, CH_COLS), jnp.float32),
            pltpu.VMEM((2, CH_ROWS, PACK * DIM), jnp.float32),
            pltpu.SemaphoreType.DMA,
            pltpu.SemaphoreType.DMA,
            pltpu.SemaphoreType.DMA,
            pltpu.SemaphoreType.DMA,
        ],
    )
    wstack = pack(*wt_list, tail)

    gather = pl.kernel(
        _gather_body,
        mesh=mesh,
        out_type=jax.ShapeDtypeStruct((NF * DIM, BATCH), jnp.float32),
        compiler_params=params,
        scratch_types=[
            pltpu.VMEM((NF, _SUB), jnp.int32),
            pltpu.VMEM((NF, _SUB), jnp.int32),
            pltpu.VMEM((2, _SUB, PACK * DIM), jnp.float32),
            pltpu.VMEM((2, DIM, _SUB), jnp.float32),
            pltpu.SemaphoreType.DMA,
            pltpu.SemaphoreType.DMA,
            pltpu.SemaphoreType.DMA,
            pltpu.SemaphoreType.DMA,
            pltpu.SemaphoreType.DMA,
        ],
    )
    out_t = gather(rowidx, off, wstack)
    return out_t.T


def kernel(f00, f01, f02, f03, f04, f05, f06, f07, f08, f09,
           f10, f11, f12, f13, f14, f15, f16, f17, f18, f19,
           f20, f21, f22, f23, f24, f25,
           W_f00, W_f01, W_f02, W_f03, W_f04, W_f05, W_f06, W_f07,
           W_f08, W_f09, W_f10, W_f11, W_f12, W_f13, W_f14, W_f15,
           W_f16, W_f17, W_f18, W_f19, W_f20, W_f21, W_f22, W_f23,
           W_f24, W_f25):
    idx = [f00, f01, f02, f03, f04, f05, f06, f07, f08, f09,
           f10, f11, f12, f13, f14, f15, f16, f17, f18, f19,
           f20, f21, f22, f23, f24, f25]
    tabs = [W_f00, W_f01, W_f02, W_f03, W_f04, W_f05, W_f06, W_f07,
            W_f08, W_f09, W_f10, W_f11, W_f12, W_f13, W_f14, W_f15,
            W_f16, W_f17, W_f18, W_f19, W_f20, W_f21, W_f22, W_f23,
            W_f24, W_f25]
    idx = [i.astype(jnp.int32) for i in idx]
    return _cat_embed(idx, tabs)
